# baseline (speedup 1.0000x reference)
import functools

import jax
import jax.numpy as jnp
import numpy as np
from jax import lax
from jax.experimental import pallas as pl
from jax.experimental.pallas import tpu as pltpu
from jax.experimental.pallas import tpu_sc as plsc

_POOL = 7
_IMG = 32.0
_B, _C, _H, _W, _T = 4, 128, 32, 32, 32
_NROIS = 256
_PTS_PER_ROI = _POOL * _POOL * _POOL
_NPTS = _NROIS * _PTS_PER_ROI
_COLS = _PTS_PER_ROI * 8
_NW = 32
_PPW = _NPTS // _NW
_CHUNK = 14
_NCHUNK = _PPW // _CHUNK
_ROWS = _CHUNK * 8
_ROI_BLK = 32


def _idxw_body(rois_ref, idx_ref, w_ref):
    rb = rois_ref.shape[0]
    shape = (rb, _COLS)
    j = lax.broadcasted_iota(jnp.int32, shape, 1)
    jz = j >> 3
    iz = jz % 7
    ix = (jz // 7) % 7
    iy = jz // 49
    ky = (j >> 2) & 1
    kx = (j >> 1) & 1
    kz = j & 1

    def col(i):
        return rois_ref[:, i:i + 1] * (1.0 / _IMG)

    y1, x1, z1 = col(0), col(1), col(2)
    y2, x2, z2 = col(3), col(4), col(5)
    b = rois_ref[:, 6:7].astype(jnp.int32)

    def axis(c1, c2, ii, kk, size):
        sizef = float(size - 1)
        scale = (c2 - c1) * sizef / float(_POOL - 1)
        coord = c1 * sizef + ii.astype(jnp.float32) * scale
        c0 = jnp.floor(coord)
        frac = coord - c0
        idx = jnp.clip(c0.astype(jnp.int32) + kk, 0, size - 1)
        wgt = jnp.where(kk == 1, frac, 1.0 - frac)
        valid = (coord >= 0.0) & (coord <= sizef)
        return idx, wgt, valid

    yi, wy, vy = axis(y1, y2, iy, ky, _H)
    xi, wx, vx = axis(x1, x2, ix, kx, _W)
    zi, wz, vz = axis(z1, z2, iz, kz, _T)
    valid = vy & vx & vz
    idx_ref[...] = ((b * _H + yi) * _W + xi) * _T + zi
    w_ref[...] = jnp.where(valid, wy * wx * wz, 0.0)


_idxw_call = pl.pallas_call(
    _idxw_body,
    grid=(_NROIS // _ROI_BLK,),
    in_specs=[pl.BlockSpec((_ROI_BLK, 7), lambda i: (i, 0))],
    out_specs=[
        pl.BlockSpec((_ROI_BLK, _COLS), lambda i: (i, 0)),
        pl.BlockSpec((_ROI_BLK, _COLS), lambda i: (i, 0)),
    ],
    out_shape=[
        jax.ShapeDtypeStruct((_NROIS, _COLS), jnp.int32),
        jax.ShapeDtypeStruct((_NROIS, _COLS), jnp.float32),
    ],
)

def _sc_interp_body(table_hbm, idx_hbm, w_hbm, out_hbm, idx_v, w_v, rows0,
                    rows1, out_v, s0, s1):
    cid = lax.axis_index("c")
    sid = lax.axis_index("s")
    wid = sid * 2 + cid
    base = wid * _PPW * _C
    pltpu.sync_copy(idx_hbm.at[wid], idx_v)
    pltpu.sync_copy(w_hbm.at[wid], w_v)

    pltpu.async_copy(table_hbm.at[idx_v.at[0]], rows0, s0)

    def compute(ci, rows_v, out_off):
        def pt_body(p, c2):
            r0 = p * 8
            wvec = w_v[ci, pl.ds(p * 16, 16)]
            ws = [wvec[k] for k in range(8)]
            ob = out_off + p * _C
            for g in range(8):
                sl = pl.ds(g * 16, 16)
                t = [ws[k] * rows_v[r0 + k, sl] for k in range(8)]
                acc = ((t[0] + t[1]) + (t[2] + t[3])) + (
                    (t[4] + t[5]) + (t[6] + t[7]))
                out_v[pl.ds(ob + g * 16, 16)] = acc
            return c2

        lax.fori_loop(0, _CHUNK, pt_body, 0, unroll=7)

    def pair_body(cj, carry):
        c0 = cj * 2
        pltpu.make_async_copy(table_hbm.at[idx_v.at[c0]], rows0, s0).wait()
        pltpu.async_copy(table_hbm.at[idx_v.at[c0 + 1]], rows1, s1)
        compute(c0, rows0, 0)
        nxt = lax.rem(c0 + 2, _NCHUNK)
        pltpu.make_async_copy(
            table_hbm.at[idx_v.at[c0 + 1]], rows1, s1).wait()
        pltpu.async_copy(table_hbm.at[idx_v.at[nxt]], rows0, s0)
        compute(c0 + 1, rows1, _CHUNK * _C)
        pltpu.sync_copy(
            out_v,
            out_hbm.at[pl.ds(base + c0 * _CHUNK * _C, 2 * _CHUNK * _C)])
        return carry

    lax.fori_loop(0, _NCHUNK // 2, pair_body, 0)
    pltpu.make_async_copy(table_hbm.at[idx_v.at[0]], rows0, s0).wait()


@functools.cache
def _sc_interp():
    mesh = plsc.VectorSubcoreMesh(core_axis_name="c", subcore_axis_name="s",
                                  num_cores=2, num_subcores=16)
    return pl.kernel(
        _sc_interp_body,
        out_type=jax.ShapeDtypeStruct((_NPTS * _C,), jnp.float32),
        mesh=mesh,
        scratch_types=[
            pltpu.VMEM((_NCHUNK, _ROWS), jnp.int32),
            pltpu.VMEM((_NCHUNK, _CHUNK * 16), jnp.float32),
            pltpu.VMEM((_ROWS, _C), jnp.float32),
            pltpu.VMEM((_ROWS, _C), jnp.float32),
            pltpu.VMEM((2 * _CHUNK * _C,), jnp.float32),
            pltpu.SemaphoreType.DMA,
            pltpu.SemaphoreType.DMA,
        ],
    )


def kernel(features, rois):
    table = jnp.transpose(features, (0, 2, 3, 4, 1)).reshape(
        _B * _H * _W * _T, _C)
    idx2d, w2d = _idxw_call(rois)
    idx3 = idx2d.reshape(_NW, _NCHUNK, _ROWS)
    wp = w2d.reshape(_NPTS, 8)
    wp = jnp.concatenate([wp, jnp.zeros((_NPTS, 8), jnp.float32)], axis=1)
    w3 = wp.reshape(_NW, _NCHUNK, _CHUNK * 16)
    out_flat = _sc_interp()(table, idx3, w3)
    return (out_flat.reshape(_NROIS, _PTS_PER_ROI, _C)
            .transpose(0, 2, 1)
            .reshape(_NROIS, _C, _POOL, _POOL, _POOL))


# speedup vs baseline: 1.4211x; 1.4211x over previous
import functools

import jax
import jax.numpy as jnp
import numpy as np
from jax import lax
from jax.experimental import pallas as pl
from jax.experimental.pallas import tpu as pltpu
from jax.experimental.pallas import tpu_sc as plsc

_POOL = 7
_IMG = 32.0
_B, _C, _H, _W, _T = 4, 128, 32, 32, 32
_NROIS = 256
_PTS_PER_ROI = _POOL * _POOL * _POOL
_NPTS = _NROIS * _PTS_PER_ROI
_COLS = _PTS_PER_ROI * 8
_NW = 32
_PPW = _NPTS // _NW
_CHUNK = 14
_NCHUNK = _PPW // _CHUNK
_ROWS = _CHUNK * 8
_ROI_BLK = 32


def _idxw_body(rois_ref, idx_ref, w_ref):
    rb = rois_ref.shape[0]
    shape = (rb, _COLS)
    j = lax.broadcasted_iota(jnp.int32, shape, 1)
    jz = j >> 3
    iz = jz % 7
    ix = (jz // 7) % 7
    iy = jz // 49
    ky = (j >> 2) & 1
    kx = (j >> 1) & 1
    kz = j & 1

    def col(i):
        return rois_ref[:, i:i + 1] * (1.0 / _IMG)

    y1, x1, z1 = col(0), col(1), col(2)
    y2, x2, z2 = col(3), col(4), col(5)
    b = rois_ref[:, 6:7].astype(jnp.int32)

    def axis(c1, c2, ii, kk, size):
        sizef = float(size - 1)
        scale = (c2 - c1) * sizef / float(_POOL - 1)
        coord = c1 * sizef + ii.astype(jnp.float32) * scale
        c0 = jnp.floor(coord)
        frac = coord - c0
        idx = jnp.clip(c0.astype(jnp.int32) + kk, 0, size - 1)
        wgt = jnp.where(kk == 1, frac, 1.0 - frac)
        valid = (coord >= 0.0) & (coord <= sizef)
        return idx, wgt, valid

    yi, wy, vy = axis(y1, y2, iy, ky, _H)
    xi, wx, vx = axis(x1, x2, ix, kx, _W)
    zi, wz, vz = axis(z1, z2, iz, kz, _T)
    valid = vy & vx & vz
    idx_ref[...] = ((b * _H + yi) * _W + xi) * _T + zi
    w_ref[...] = jnp.where(valid, wy * wx * wz, 0.0)


_idxw_call = pl.pallas_call(
    _idxw_body,
    grid=(_NROIS // _ROI_BLK,),
    in_specs=[pl.BlockSpec((_ROI_BLK, 7), lambda i: (i, 0))],
    out_specs=[
        pl.BlockSpec((_ROI_BLK, _COLS), lambda i: (i, 0)),
        pl.BlockSpec((_ROI_BLK, _COLS), lambda i: (i, 0)),
    ],
    out_shape=[
        jax.ShapeDtypeStruct((_NROIS, _COLS), jnp.int32),
        jax.ShapeDtypeStruct((_NROIS, _COLS), jnp.float32),
    ],
)

def _sc_interp_body(table_hbm, idx_hbm, w_hbm, out_hbm, idx_v, w_v, rows0,
                    rows1, out_v, s0, s1):
    cid = lax.axis_index("c")
    sid = lax.axis_index("s")
    wid = sid * 2 + cid
    base = wid * _PPW * _C
    pltpu.sync_copy(idx_hbm.at[wid], idx_v)
    pltpu.sync_copy(w_hbm.at[wid], w_v)

    pltpu.async_copy(table_hbm.at[idx_v.at[0]], rows0, s0)

    def compute(ci, rows_v, out_off):
        def pair_body2(q, c2):
            p0 = q * 2
            r0 = p0 * 8
            wvec = w_v[ci, pl.ds(r0, 16)]
            for half in range(2):
                ws = [wvec[half * 8 + k] for k in range(8)]
                rr = r0 + half * 8
                ob = out_off + (p0 + half) * _C
                for g in range(8):
                    sl = pl.ds(g * 16, 16)
                    t = [ws[k] * rows_v[rr + k, sl] for k in range(8)]
                    acc = ((t[0] + t[1]) + (t[2] + t[3])) + (
                        (t[4] + t[5]) + (t[6] + t[7]))
                    out_v[pl.ds(ob + g * 16, 16)] = acc
            return c2

        lax.fori_loop(0, 7, pair_body2, 0)

    def pair_body(cj, carry):
        c0 = cj * 2
        pltpu.make_async_copy(table_hbm.at[idx_v.at[c0]], rows0, s0).wait()
        pltpu.async_copy(table_hbm.at[idx_v.at[c0 + 1]], rows1, s1)
        compute(c0, rows0, 0)
        nxt = lax.rem(c0 + 2, _NCHUNK)
        pltpu.make_async_copy(
            table_hbm.at[idx_v.at[c0 + 1]], rows1, s1).wait()
        pltpu.async_copy(table_hbm.at[idx_v.at[nxt]], rows0, s0)
        compute(c0 + 1, rows1, _CHUNK * _C)
        pltpu.sync_copy(
            out_v,
            out_hbm.at[pl.ds(base + c0 * _CHUNK * _C, 2 * _CHUNK * _C)])
        return carry

    lax.fori_loop(0, _NCHUNK // 2, pair_body, 0)
    pltpu.make_async_copy(table_hbm.at[idx_v.at[0]], rows0, s0).wait()


@functools.cache
def _sc_interp():
    mesh = plsc.VectorSubcoreMesh(core_axis_name="c", subcore_axis_name="s",
                                  num_cores=2, num_subcores=16)
    return pl.kernel(
        _sc_interp_body,
        out_type=jax.ShapeDtypeStruct((_NPTS * _C,), jnp.float32),
        mesh=mesh,
        scratch_types=[
            pltpu.VMEM((_NCHUNK, _ROWS), jnp.int32),
            pltpu.VMEM((_NCHUNK, _ROWS), jnp.float32),
            pltpu.VMEM((_ROWS, _C), jnp.float32),
            pltpu.VMEM((_ROWS, _C), jnp.float32),
            pltpu.VMEM((2 * _CHUNK * _C,), jnp.float32),
            pltpu.SemaphoreType.DMA,
            pltpu.SemaphoreType.DMA,
        ],
    )


def kernel(features, rois):
    table = jnp.transpose(features, (0, 2, 3, 4, 1)).reshape(
        _B * _H * _W * _T, _C)
    idx2d, w2d = _idxw_call(rois)
    idx3 = idx2d.reshape(_NW, _NCHUNK, _ROWS)
    w3 = w2d.reshape(_NW, _NCHUNK, _ROWS)
    out_flat = _sc_interp()(table, idx3, w3)
    return (out_flat.reshape(_NROIS, _PTS_PER_ROI, _C)
            .transpose(0, 2, 1)
            .reshape(_NROIS, _C, _POOL, _POOL, _POOL))


# speedup vs baseline: 1.5787x; 1.1109x over previous
import functools

import jax
import jax.numpy as jnp
import numpy as np
from jax import lax
from jax.experimental import pallas as pl
from jax.experimental.pallas import tpu as pltpu
from jax.experimental.pallas import tpu_sc as plsc

_POOL = 7
_IMG = 32.0
_B, _C, _H, _W, _T = 4, 128, 32, 32, 32
_NROIS = 256
_PTS_PER_ROI = _POOL * _POOL * _POOL
_NPTS = _NROIS * _PTS_PER_ROI
_COLS = _PTS_PER_ROI * 8
_NW = 32
_PPW = _NPTS // _NW
_CHUNK = 14
_NCHUNK = _PPW // _CHUNK
_ROWS = _CHUNK * 8
_ROI_BLK = 32


def _idxw_body(rois_ref, idx_ref, w_ref):
    rb = rois_ref.shape[0]
    shape = (rb, _COLS)
    j = lax.broadcasted_iota(jnp.int32, shape, 1)
    jz = j >> 3
    iz = jz % 7
    ix = (jz // 7) % 7
    iy = jz // 49
    ky = (j >> 2) & 1
    kx = (j >> 1) & 1
    kz = j & 1

    def col(i):
        return rois_ref[:, i:i + 1] * (1.0 / _IMG)

    y1, x1, z1 = col(0), col(1), col(2)
    y2, x2, z2 = col(3), col(4), col(5)
    b = rois_ref[:, 6:7].astype(jnp.int32)

    def axis(c1, c2, ii, kk, size):
        sizef = float(size - 1)
        scale = (c2 - c1) * sizef / float(_POOL - 1)
        coord = c1 * sizef + ii.astype(jnp.float32) * scale
        c0 = jnp.floor(coord)
        frac = coord - c0
        idx = jnp.clip(c0.astype(jnp.int32) + kk, 0, size - 1)
        wgt = jnp.where(kk == 1, frac, 1.0 - frac)
        valid = (coord >= 0.0) & (coord <= sizef)
        return idx, wgt, valid

    yi, wy, vy = axis(y1, y2, iy, ky, _H)
    xi, wx, vx = axis(x1, x2, ix, kx, _W)
    zi, wz, vz = axis(z1, z2, iz, kz, _T)
    valid = vy & vx & vz
    idx_ref[...] = ((b * _H + yi) * _W + xi) * _T + zi
    w_ref[...] = jnp.where(valid, wy * wx * wz, 0.0)


_idxw_call = pl.pallas_call(
    _idxw_body,
    grid=(_NROIS // _ROI_BLK,),
    in_specs=[pl.BlockSpec((_ROI_BLK, 7), lambda i: (i, 0))],
    out_specs=[
        pl.BlockSpec((_ROI_BLK, _COLS), lambda i: (i, 0)),
        pl.BlockSpec((_ROI_BLK, _COLS), lambda i: (i, 0)),
    ],
    out_shape=[
        jax.ShapeDtypeStruct((_NROIS, _COLS), jnp.int32),
        jax.ShapeDtypeStruct((_NROIS, _COLS), jnp.float32),
    ],
)

def _sc_interp_body(table_hbm, idx_hbm, w_hbm, out_hbm, idx_v, w_v, rows0,
                    rows1, rows2, rows3, out_v, s0, s1, s2, s3):
    cid = lax.axis_index("c")
    sid = lax.axis_index("s")
    wid = sid * 2 + cid
    base = wid * _PPW * _C
    pltpu.sync_copy(idx_hbm.at[wid], idx_v)
    pltpu.sync_copy(w_hbm.at[wid], w_v)

    pltpu.async_copy(table_hbm.at[idx_v.at[0]], rows0, s0)
    pltpu.async_copy(table_hbm.at[idx_v.at[1]], rows1, s1)
    pltpu.async_copy(table_hbm.at[idx_v.at[2]], rows2, s2)
    pltpu.async_copy(table_hbm.at[idx_v.at[3]], rows3, s3)

    def compute(ci, rows_v, out_off):
        def pair_body2(q, c2):
            p0 = q * 2
            r0 = p0 * 8
            wvec = w_v[ci, pl.ds(r0, 16)]
            for half in range(2):
                ws = [wvec[half * 8 + k] for k in range(8)]
                rr = r0 + half * 8
                ob = out_off + (p0 + half) * _C
                for g in range(8):
                    sl = pl.ds(g * 16, 16)
                    t = [ws[k] * rows_v[rr + k, sl] for k in range(8)]
                    acc = ((t[0] + t[1]) + (t[2] + t[3])) + (
                        (t[4] + t[5]) + (t[6] + t[7]))
                    out_v[pl.ds(ob + g * 16, 16)] = acc
            return c2

        lax.fori_loop(0, 7, pair_body2, 0)

    def quad_body(cj, carry):
        c0 = cj * 4
        for j, (rv, sv) in enumerate(
                ((rows0, s0), (rows1, s1), (rows2, s2), (rows3, s3))):
            ci = c0 + j
            pltpu.make_async_copy(table_hbm.at[idx_v.at[ci]], rv, sv).wait()
            compute(ci, rv, j * _CHUNK * _C)

            @pl.when(ci + 4 < _NCHUNK)
            def _():
                pltpu.async_copy(table_hbm.at[idx_v.at[ci + 4]], rv, sv)

        pltpu.sync_copy(
            out_v,
            out_hbm.at[pl.ds(base + c0 * _CHUNK * _C, 4 * _CHUNK * _C)])
        return carry

    lax.fori_loop(0, _NCHUNK // 4, quad_body, 0)


@functools.cache
def _sc_interp():
    mesh = plsc.VectorSubcoreMesh(core_axis_name="c", subcore_axis_name="s",
                                  num_cores=2, num_subcores=16)
    return pl.kernel(
        _sc_interp_body,
        out_type=jax.ShapeDtypeStruct((_NPTS * _C,), jnp.float32),
        mesh=mesh,
        scratch_types=[
            pltpu.VMEM((_NCHUNK, _ROWS), jnp.int32),
            pltpu.VMEM((_NCHUNK, _ROWS), jnp.float32),
            pltpu.VMEM((_ROWS, _C), jnp.float32),
            pltpu.VMEM((_ROWS, _C), jnp.float32),
            pltpu.VMEM((_ROWS, _C), jnp.float32),
            pltpu.VMEM((_ROWS, _C), jnp.float32),
            pltpu.VMEM((4 * _CHUNK * _C,), jnp.float32),
            pltpu.SemaphoreType.DMA,
            pltpu.SemaphoreType.DMA,
            pltpu.SemaphoreType.DMA,
            pltpu.SemaphoreType.DMA,
        ],
    )


def kernel(features, rois):
    table = jnp.transpose(features, (0, 2, 3, 4, 1)).reshape(
        _B * _H * _W * _T, _C)
    idx2d, w2d = _idxw_call(rois)
    idx3 = idx2d.reshape(_NW, _NCHUNK, _ROWS)
    w3 = w2d.reshape(_NW, _NCHUNK, _ROWS)
    out_flat = _sc_interp()(table, idx3, w3)
    return (out_flat.reshape(_NROIS, _PTS_PER_ROI, _C)
            .transpose(0, 2, 1)
            .reshape(_NROIS, _C, _POOL, _POOL, _POOL))


# speedup vs baseline: 1.5803x; 1.0010x over previous
import functools

import jax
import jax.numpy as jnp
from jax import lax
from jax.experimental import pallas as pl
from jax.experimental.pallas import tpu as pltpu
from jax.experimental.pallas import tpu_sc as plsc

_POOL = 7
_IMG = 32.0
_B, _C, _H, _W, _T = 4, 128, 32, 32, 32
_NROIS = 256
_PTS_PER_ROI = _POOL * _POOL * _POOL
_NPTS = _NROIS * _PTS_PER_ROI
_COLS = _PTS_PER_ROI * 8
_NW = 32
_PPW = _NPTS // _NW
_CHUNK = 14
_NCHUNK = _PPW // _CHUNK
_ROWS = _CHUNK * 8
_ROI_BLK = 32


def _idxw_body(rois_ref, idx_ref, w_ref):
    rb = rois_ref.shape[0]
    shape = (rb, _COLS)
    j = lax.broadcasted_iota(jnp.int32, shape, 1)
    jz = j >> 3
    iz = jz % 7
    ix = (jz // 7) % 7
    iy = jz // 49
    ky = (j >> 2) & 1
    kx = (j >> 1) & 1
    kz = j & 1

    def col(i):
        return rois_ref[:, i:i + 1] * (1.0 / _IMG)

    y1, x1, z1 = col(0), col(1), col(2)
    y2, x2, z2 = col(3), col(4), col(5)
    b = rois_ref[:, 6:7].astype(jnp.int32)

    def axis(c1, c2, ii, kk, size):
        sizef = float(size - 1)
        scale = (c2 - c1) * sizef / float(_POOL - 1)
        coord = c1 * sizef + ii.astype(jnp.float32) * scale
        c0 = jnp.floor(coord)
        frac = coord - c0
        idx = jnp.clip(c0.astype(jnp.int32) + kk, 0, size - 1)
        wgt = jnp.where(kk == 1, frac, 1.0 - frac)
        valid = (coord >= 0.0) & (coord <= sizef)
        return idx, wgt, valid

    yi, wy, vy = axis(y1, y2, iy, ky, _H)
    xi, wx, vx = axis(x1, x2, ix, kx, _W)
    zi, wz, vz = axis(z1, z2, iz, kz, _T)
    valid = vy & vx & vz
    idx_ref[...] = ((b * _H + yi) * _W + xi) * _T + zi
    w_ref[...] = jnp.where(valid, wy * wx * wz, 0.0)


_idxw_call = pl.pallas_call(
    _idxw_body,
    grid=(_NROIS // _ROI_BLK,),
    in_specs=[pl.BlockSpec((_ROI_BLK, 7), lambda i: (i, 0))],
    out_specs=[
        pl.BlockSpec((_ROI_BLK, _COLS), lambda i: (i, 0)),
        pl.BlockSpec((_ROI_BLK, _COLS), lambda i: (i, 0)),
    ],
    out_shape=[
        jax.ShapeDtypeStruct((_NROIS, _COLS), jnp.int32),
        jax.ShapeDtypeStruct((_NROIS, _COLS), jnp.float32),
    ],
)

def _sc_interp_body(table_hbm, idx_hbm, w_hbm, out_hbm, idx_v, w_v, rows0,
                    rows1, rows2, rows3, out_v, s0, s1, s2, s3):
    cid = lax.axis_index("c")
    sid = lax.axis_index("s")
    wid = sid * 2 + cid
    base = wid * _PPW * _C
    pltpu.sync_copy(idx_hbm.at[wid], idx_v)
    pltpu.sync_copy(w_hbm.at[wid], w_v)

    pltpu.async_copy(table_hbm.at[idx_v.at[0]], rows0, s0)
    pltpu.async_copy(table_hbm.at[idx_v.at[1]], rows1, s1)
    pltpu.async_copy(table_hbm.at[idx_v.at[2]], rows2, s2)
    pltpu.async_copy(table_hbm.at[idx_v.at[3]], rows3, s3)

    def compute(ci, rows_v, out_off):
        def pair_body2(q, c2):
            p0 = q * 2
            r0 = p0 * 8
            wvec = w_v[ci, pl.ds(r0, 16)]
            for half in range(2):
                ws = [wvec[half * 8 + k] for k in range(8)]
                rr = r0 + half * 8
                ob = out_off + (p0 + half) * _C
                for g in range(8):
                    sl = pl.ds(g * 16, 16)
                    t = [ws[k] * rows_v[rr + k, sl] for k in range(8)]
                    acc = ((t[0] + t[1]) + (t[2] + t[3])) + (
                        (t[4] + t[5]) + (t[6] + t[7]))
                    out_v[pl.ds(ob + g * 16, 16)] = acc
            return c2

        lax.fori_loop(0, 7, pair_body2, 0)

    def quad_body(cj, carry):
        c0 = cj * 4
        for j, (rv, sv) in enumerate(
                ((rows0, s0), (rows1, s1), (rows2, s2), (rows3, s3))):
            ci = c0 + j
            pltpu.make_async_copy(table_hbm.at[idx_v.at[ci]], rv, sv).wait()
            compute(ci, rv, j * _CHUNK * _C)

            @pl.when(ci + 4 < _NCHUNK)
            def _():
                pltpu.async_copy(table_hbm.at[idx_v.at[ci + 4]], rv, sv)

        pltpu.sync_copy(
            out_v,
            out_hbm.at[pl.ds(base + c0 * _CHUNK * _C, 4 * _CHUNK * _C)])
        return carry

    lax.fori_loop(0, _NCHUNK // 4, quad_body, 0)


@functools.cache
def _sc_interp():
    mesh = plsc.VectorSubcoreMesh(core_axis_name="c", subcore_axis_name="s",
                                  num_cores=2, num_subcores=16)
    return pl.kernel(
        _sc_interp_body,
        out_type=jax.ShapeDtypeStruct((_NPTS * _C,), jnp.float32),
        mesh=mesh,
        scratch_types=[
            pltpu.VMEM((_NCHUNK, _ROWS), jnp.int32),
            pltpu.VMEM((_NCHUNK, _ROWS), jnp.float32),
            pltpu.VMEM((_ROWS, _C), jnp.float32),
            pltpu.VMEM((_ROWS, _C), jnp.float32),
            pltpu.VMEM((_ROWS, _C), jnp.float32),
            pltpu.VMEM((_ROWS, _C), jnp.float32),
            pltpu.VMEM((4 * _CHUNK * _C,), jnp.float32),
            pltpu.SemaphoreType.DMA,
            pltpu.SemaphoreType.DMA,
            pltpu.SemaphoreType.DMA,
            pltpu.SemaphoreType.DMA,
        ],
    )


def kernel(features, rois):
    table = jnp.transpose(features, (0, 2, 3, 4, 1)).reshape(
        _B * _H * _W * _T, _C)
    idx2d, w2d = _idxw_call(rois)
    idx3 = idx2d.reshape(_NW, _NCHUNK, _ROWS)
    w3 = w2d.reshape(_NW, _NCHUNK, _ROWS)
    out_flat = _sc_interp()(table, idx3, w3)
    return (out_flat.reshape(_NROIS, _PTS_PER_ROI, _C)
            .transpose(0, 2, 1)
            .reshape(_NROIS, _C, _POOL, _POOL, _POOL))
